# baseline (speedup 1.0000x reference)
import dataclasses
import functools

import jax
import jax.numpy as jnp
from jax import lax
from jax.experimental import pallas as pl
from jax.experimental.pallas import tpu as pltpu
from jax.experimental.pallas import tpu_sc as plsc

N = 10000
E = 160000
D_IN = 256
D_HID = 256
D_OUT = 128

NC = 2
NS = 16
CHUNK = 128
NCHUNKS = E // CHUNK
HALF = NCHUNKS // 2
RC = 128
NRC = N // RC
NREM = N - NRC * RC
NRLOOP = (NRC + NS - 1) // NS

_mesh = functools.partial(
    plsc.VectorSubcoreMesh, core_axis_name="c", subcore_axis_name="s")


def _zero_vec(ref, n):
    z = jnp.zeros((16,), jnp.float32)

    @pl.loop(0, n // 16)
    def _(i):
        ref[pl.ds(i * 16, 16)] = z


def _fill_ones(ref, n):
    o = jnp.ones((16,), jnp.float32)

    @pl.loop(0, n // 16)
    def _(i):
        ref[pl.ds(i * 16, 16)] = o


def _zero_rows(ref):
    z = jnp.zeros((16,), jnp.float32)

    @pl.loop(0, RC)
    def _(r):
        @pl.loop(0, 8)
        def _(c):
            ref[r, pl.ds(c * 16, 16)] = z


def _rows_phase(sid, fn):
    @pl.loop(0, NRLOOP)
    def _(it):
        rc = sid + it * NS

        @pl.when(rc < NRC)
        def _():
            fn(rc * RC, RC)

    @pl.when(sid == 0)
    def _():
        fn(NRC * RC, NREM)


def _edge_pipeline(cid, sid, mode, start_idx, wait_idx, start_body,
                   finish_body):
    if mode == "feat":
        nloop = (NCHUNKS + NS - 1) // NS

        def chunk_of(j):
            return sid + j * NS

        def limit():
            return NCHUNKS
    else:
        nloop = (HALF + NS - 1) // NS

        def chunk_of(j):
            return cid * HALF + sid + j * NS

        def limit():
            return (cid + 1) * HALF

    npair = (nloop + 1) // 2

    for b in (0, 1):
        @pl.when(chunk_of(b) < limit())
        def _(b=b):
            start_idx(b, b)

    @pl.loop(0, npair)
    def _(j2):
        j0 = j2 * 2
        for b in (0, 1):
            j = j0 + b

            @pl.when(chunk_of(j) < limit())
            def _(b=b, j=j):
                wait_idx(j, b)
                start_body(j, b)
        for b in (0, 1):
            j = j0 + b

            @pl.when(chunk_of(j) < limit())
            def _(b=b, j=j):
                finish_body(j, b)

                @pl.when(chunk_of(j + 2) < limit())
                def _():
                    start_idx(j + 2, b)

    return chunk_of


def _sc_histogram(dst):
    out_type = (jax.ShapeDtypeStruct((N,), jnp.float32),
                jax.ShapeDtypeStruct((N,), jnp.float32))

    @functools.partial(
        pl.kernel,
        out_type=out_type,
        mesh=_mesh(),
        scratch_types=[
            pltpu.VMEM((CHUNK,), jnp.float32),
            pltpu.VMEM((CHUNK,), jnp.int32),
            pltpu.VMEM((CHUNK,), jnp.int32),
            pltpu.VMEM((2000,), jnp.float32),
            pltpu.VMEM((N,), jnp.float32),
            pltpu.SemaphoreType.DMA,
            pltpu.SemaphoreType.DMA,
            pltpu.VMEM_SHARED((N,), jnp.float32),
        ],
    )
    def k(dst_hbm, cnt_a, cnt_b, ones_v, idx0, idx1, zvec, stage,
          s0, s1, acc_s):
        cid = lax.axis_index("c")
        sid = lax.axis_index("s")
        idx = (idx0, idx1)
        sem = (s0, s1)

        _fill_ones(ones_v, CHUNK)

        @pl.when(sid == 0)
        def _():
            _zero_vec(zvec, 2000)

            @pl.loop(0, 5)
            def _(j):
                pltpu.sync_copy(zvec, acc_s.at[pl.ds(j * 2000, 2000)])

        plsc.subcore_barrier()

        def start_idx(j, b):
            base = (cid * HALF + sid + j * NS) * CHUNK
            pltpu.async_copy(dst_hbm.at[pl.ds(base, CHUNK)], idx[b], sem[b])

        def wait_idx(j, b):
            base = (cid * HALF + sid + j * NS) * CHUNK
            pltpu.make_async_copy(dst_hbm.at[pl.ds(base, CHUNK)], idx[b],
                                  sem[b]).wait()

        def start_body(j, b):
            pass

        def finish_body(j, b):
            pltpu.sync_copy(ones_v, acc_s.at[idx[b]], add=True)

        _edge_pipeline(cid, sid, "edge", start_idx, wait_idx, start_body,
                       finish_body)

        plsc.subcore_barrier()

        @pl.when(sid == 0)
        def _():
            pltpu.sync_copy(acc_s, stage)

            @pl.when(cid == 0)
            def _():
                pltpu.sync_copy(stage, cnt_a)

            @pl.when(cid == 1)
            def _():
                pltpu.sync_copy(stage, cnt_b)

    return k(dst)


def _sc_scatter_rows(tables, src, dst, mode):
    out_type = tuple(jax.ShapeDtypeStruct((N, 128), jnp.float32)
                     for _ in range(2))

    @functools.partial(
        pl.kernel,
        out_type=out_type,
        mesh=_mesh(),
        scratch_types=[
            pltpu.VMEM((CHUNK,), jnp.int32),
            pltpu.VMEM((CHUNK,), jnp.int32),
            pltpu.VMEM((CHUNK,), jnp.int32),
            pltpu.VMEM((CHUNK,), jnp.int32),
            pltpu.VMEM((CHUNK, 128), jnp.float32),
            pltpu.VMEM((CHUNK, 128), jnp.float32),
            pltpu.SemaphoreType.DMA,
            pltpu.SemaphoreType.DMA,
            pltpu.SemaphoreType.DMA,
            pltpu.SemaphoreType.DMA,
            pltpu.SemaphoreType.DMA,
            pltpu.SemaphoreType.DMA,
            pltpu.VMEM_SHARED((N, 128), jnp.float32),
        ],
    )
    def k(*refs):
        if mode == "feat":
            ta, tb, src_hbm, dst_hbm, out_a, out_b = refs[:6]
            nin = 6
        else:
            tab, src_hbm, dst_hbm, out_a, out_b = refs[:5]
            nin = 5
        (sidx0, sidx1, didx0, didx1, rows0, rows1,
         ss0, ss1, sd0, sd1, sg0, sg1, acc_s) = refs[nin:]
        sidx = (sidx0, sidx1)
        didx = (didx0, didx1)
        rows = (rows0, rows1)
        ssem = (ss0, ss1)
        dsem = (sd0, sd1)
        gsem = (sg0, sg1)
        cid = lax.axis_index("c")
        sid = lax.axis_index("s")

        _zero_rows(rows0)
        _rows_phase(sid, lambda r0, nr: pltpu.sync_copy(
            rows0.at[pl.ds(0, nr)], acc_s.at[pl.ds(r0, nr)]))

        plsc.subcore_barrier()

        if mode == "feat":
            def base_of(j):
                return (sid + j * NS) * CHUNK
        else:
            def base_of(j):
                return (cid * HALF + sid + j * NS) * CHUNK

        def start_idx(j, b):
            base = base_of(j)
            pltpu.async_copy(src_hbm.at[pl.ds(base, CHUNK)], sidx[b],
                             ssem[b])
            pltpu.async_copy(dst_hbm.at[pl.ds(base, CHUNK)], didx[b],
                             dsem[b])

        def wait_idx(j, b):
            base = base_of(j)
            pltpu.make_async_copy(src_hbm.at[pl.ds(base, CHUNK)], sidx[b],
                                  ssem[b]).wait()
            pltpu.make_async_copy(dst_hbm.at[pl.ds(base, CHUNK)], didx[b],
                                  dsem[b]).wait()

        def start_body(j, b):
            if mode == "feat":
                @pl.when(cid == 0)
                def _():
                    pltpu.async_copy(ta.at[sidx[b]], rows[b], gsem[b])

                @pl.when(cid == 1)
                def _():
                    pltpu.async_copy(tb.at[sidx[b]], rows[b], gsem[b])
            else:
                pltpu.async_copy(tab.at[sidx[b]], rows[b], gsem[b])

        def finish_body(j, b):
            first = ta if mode == "feat" else tab
            pltpu.make_async_copy(first.at[sidx[b]], rows[b],
                                  gsem[b]).wait()
            pltpu.sync_copy(rows[b], acc_s.at[didx[b]], add=True)

        _edge_pipeline(cid, sid, mode, start_idx, wait_idx, start_body,
                       finish_body)

        plsc.subcore_barrier()

        def wb(r0, nr):
            pltpu.sync_copy(acc_s.at[pl.ds(r0, nr)], rows0.at[pl.ds(0, nr)])

            @pl.when(cid == 0)
            def _():
                pltpu.sync_copy(rows0.at[pl.ds(0, nr)],
                                out_a.at[pl.ds(r0, nr)])

            @pl.when(cid == 1)
            def _():
                pltpu.sync_copy(rows0.at[pl.ds(0, nr)],
                                out_b.at[pl.ds(r0, nr)])

        _rows_phase(sid, wb)

    if mode == "feat":
        return k(tables[0], tables[1], src, dst)
    return k(tables[0], src, dst)


def _sc_scores(q, kk, src, dst):
    @functools.partial(
        pl.kernel,
        out_type=jax.ShapeDtypeStruct((E, 16), jnp.float32),
        mesh=_mesh(),
        scratch_types=[
            pltpu.VMEM((CHUNK,), jnp.int32),
            pltpu.VMEM((CHUNK,), jnp.int32),
            pltpu.VMEM((CHUNK,), jnp.int32),
            pltpu.VMEM((CHUNK,), jnp.int32),
            pltpu.VMEM((CHUNK, 128), jnp.float32),
            pltpu.VMEM((CHUNK, 128), jnp.float32),
            pltpu.VMEM((CHUNK, 128), jnp.float32),
            pltpu.VMEM((CHUNK, 128), jnp.float32),
            pltpu.VMEM((CHUNK, 16), jnp.float32),
            pltpu.VMEM((CHUNK, 16), jnp.float32),
            pltpu.SemaphoreType.DMA,
            pltpu.SemaphoreType.DMA,
            pltpu.SemaphoreType.DMA,
            pltpu.SemaphoreType.DMA,
            pltpu.SemaphoreType.DMA,
            pltpu.SemaphoreType.DMA,
            pltpu.SemaphoreType.DMA,
            pltpu.SemaphoreType.DMA,
        ],
    )
    def k(q_hbm, k_hbm, src_hbm, dst_hbm, sc_hbm,
          sidx0, sidx1, didx0, didx1, rq0, rq1, rk0, rk1, sb0, sb1,
          ss0, ss1, sd0, sd1, sq0, sq1, sk0, sk1):
        sidx = (sidx0, sidx1)
        didx = (didx0, didx1)
        rq = (rq0, rq1)
        rk = (rk0, rk1)
        sbuf = (sb0, sb1)
        ssem = (ss0, ss1)
        dsem = (sd0, sd1)
        qsem = (sq0, sq1)
        ksem = (sk0, sk1)
        cid = lax.axis_index("c")
        sid = lax.axis_index("s")

        def base_of(j):
            return (cid * HALF + sid + j * NS) * CHUNK

        def start_idx(j, b):
            base = base_of(j)
            pltpu.async_copy(src_hbm.at[pl.ds(base, CHUNK)], sidx[b],
                             ssem[b])
            pltpu.async_copy(dst_hbm.at[pl.ds(base, CHUNK)], didx[b],
                             dsem[b])

        def wait_idx(j, b):
            base = base_of(j)
            pltpu.make_async_copy(src_hbm.at[pl.ds(base, CHUNK)], sidx[b],
                                  ssem[b]).wait()
            pltpu.make_async_copy(dst_hbm.at[pl.ds(base, CHUNK)], didx[b],
                                  dsem[b]).wait()

        def start_body(j, b):
            pltpu.async_copy(q_hbm.at[didx[b]], rq[b], qsem[b])
            pltpu.async_copy(k_hbm.at[sidx[b]], rk[b], ksem[b])

        def finish_body(j, b):
            base = base_of(j)
            pltpu.make_async_copy(q_hbm.at[didx[b]], rq[b], qsem[b]).wait()
            pltpu.make_async_copy(k_hbm.at[sidx[b]], rk[b], ksem[b]).wait()

            @pl.loop(0, CHUNK)
            def _(r):
                acc = rq[b][r, pl.ds(0, 16)] * rk[b][r, pl.ds(0, 16)]
                for c in range(1, 8):
                    sl = pl.ds(c * 16, 16)
                    acc = acc + rq[b][r, sl] * rk[b][r, sl]
                sbuf[b][r, pl.ds(0, 16)] = acc

            pltpu.sync_copy(sbuf[b], sc_hbm.at[pl.ds(base, CHUNK)])

        _edge_pipeline(cid, sid, "edge", start_idx, wait_idx, start_body,
                       finish_body)

    return k(q, kk, src, dst)


def _sc_attn_scatter(v, scores, gmax, src, dst):
    out_type = (jax.ShapeDtypeStruct((N, 128), jnp.float32),
                jax.ShapeDtypeStruct((N, 128), jnp.float32),
                jax.ShapeDtypeStruct((N,), jnp.float32),
                jax.ShapeDtypeStruct((N,), jnp.float32))

    @functools.partial(
        pl.kernel,
        out_type=out_type,
        mesh=_mesh(),
        scratch_types=[
            pltpu.VMEM((CHUNK,), jnp.int32),
            pltpu.VMEM((CHUNK,), jnp.int32),
            pltpu.VMEM((CHUNK,), jnp.int32),
            pltpu.VMEM((CHUNK,), jnp.int32),
            pltpu.VMEM((CHUNK, 128), jnp.float32),
            pltpu.VMEM((CHUNK, 128), jnp.float32),
            pltpu.VMEM((CHUNK,), jnp.float32),
            pltpu.VMEM((CHUNK,), jnp.float32),
            pltpu.VMEM((128,), jnp.float32),
            pltpu.VMEM((2000,), jnp.float32),
            pltpu.SemaphoreType.DMA,
            pltpu.SemaphoreType.DMA,
            pltpu.SemaphoreType.DMA,
            pltpu.SemaphoreType.DMA,
            pltpu.SemaphoreType.DMA,
            pltpu.SemaphoreType.DMA,
            pltpu.SemaphoreType.DMA,
            pltpu.SemaphoreType.DMA,
            pltpu.VMEM_SHARED((N, 128), jnp.float32),
            pltpu.VMEM_SHARED((N,), jnp.float32),
        ],
    )
    def k(v_hbm, sc_hbm, gmax_hbm, src_hbm, dst_hbm,
          agg_a, agg_b, den_a, den_b,
          sidx0, sidx1, didx0, didx1, rows0, rows1, ev0, ev1, gbuf, zvec,
          ss0, ss1, sd0, sd1, sv0, sv1, sg0, sg1, acc_s, den_s):
        sidx = (sidx0, sidx1)
        didx = (didx0, didx1)
        rows = (rows0, rows1)
        ev = (ev0, ev1)
        ssem = (ss0, ss1)
        dsem = (sd0, sd1)
        vsem = (sv0, sv1)
        gsem = (sg0, sg1)
        cid = lax.axis_index("c")
        sid = lax.axis_index("s")

        pltpu.sync_copy(gmax_hbm, gbuf)

        _zero_rows(rows0)
        _rows_phase(sid, lambda r0, nr: pltpu.sync_copy(
            rows0.at[pl.ds(0, nr)], acc_s.at[pl.ds(r0, nr)]))

        @pl.when(sid == 0)
        def _():
            _zero_vec(zvec, 2000)

            @pl.loop(0, 5)
            def _(j):
                pltpu.sync_copy(zvec, den_s.at[pl.ds(j * 2000, 2000)])

        plsc.subcore_barrier()

        def base_of(j):
            return (cid * HALF + sid + j * NS) * CHUNK

        def start_idx(j, b):
            base = base_of(j)
            pltpu.async_copy(src_hbm.at[pl.ds(base, CHUNK)], sidx[b],
                             ssem[b])
            pltpu.async_copy(dst_hbm.at[pl.ds(base, CHUNK)], didx[b],
                             dsem[b])
            pltpu.async_copy(sc_hbm.at[pl.ds(base, CHUNK)], ev[b], vsem[b])

        def wait_idx(j, b):
            base = base_of(j)
            pltpu.make_async_copy(src_hbm.at[pl.ds(base, CHUNK)], sidx[b],
                                  ssem[b]).wait()
            pltpu.make_async_copy(dst_hbm.at[pl.ds(base, CHUNK)], didx[b],
                                  dsem[b]).wait()
            pltpu.make_async_copy(sc_hbm.at[pl.ds(base, CHUNK)], ev[b],
                                  vsem[b]).wait()

        def start_body(j, b):
            pltpu.async_copy(v_hbm.at[sidx[b]], rows[b], gsem[b])

        def finish_body(j, b):
            gm = gbuf[pl.ds(0, 16)][0]
            @pl.loop(0, CHUNK // 16)
            def _(i):
                s = ev[b][pl.ds(i * 16, 16)]
                ev[b][pl.ds(i * 16, 16)] = jnp.exp(s - gm)

            pltpu.make_async_copy(v_hbm.at[sidx[b]], rows[b],
                                  gsem[b]).wait()

            @pl.loop(0, CHUNK // 16)
            def _(g):
                evec = ev[b][pl.ds(g * 16, 16)]
                for kk_ in range(16):
                    er = evec[kk_]
                    r = g * 16 + kk_
                    for c in range(8):
                        sl = pl.ds(c * 16, 16)
                        rows[b][r, sl] = rows[b][r, sl] * er

            pltpu.sync_copy(rows[b], acc_s.at[didx[b]], add=True)
            pltpu.sync_copy(ev[b], den_s.at[didx[b]], add=True)

        _edge_pipeline(cid, sid, "edge", start_idx, wait_idx, start_body,
                       finish_body)

        plsc.subcore_barrier()

        def wb(r0, nr):
            pltpu.sync_copy(acc_s.at[pl.ds(r0, nr)], rows0.at[pl.ds(0, nr)])

            @pl.when(cid == 0)
            def _():
                pltpu.sync_copy(rows0.at[pl.ds(0, nr)],
                                agg_a.at[pl.ds(r0, nr)])

            @pl.when(cid == 1)
            def _():
                pltpu.sync_copy(rows0.at[pl.ds(0, nr)],
                                agg_b.at[pl.ds(r0, nr)])

        _rows_phase(sid, wb)

        @pl.when(sid == 0)
        def _():
            @pl.loop(0, 5)
            def _(j):
                pltpu.sync_copy(den_s.at[pl.ds(j * 2000, 2000)], zvec)

                @pl.when(cid == 0)
                def _():
                    pltpu.sync_copy(zvec, den_a.at[pl.ds(j * 2000, 2000)])

                @pl.when(cid == 1)
                def _():
                    pltpu.sync_copy(zvec, den_b.at[pl.ds(j * 2000, 2000)])

    return k(v, scores, gmax, src, dst)


ROWS = 1000


def _tc_matmul(x, W):
    n, din = x.shape
    dout = W.shape[1]

    def body(x_ref, w_ref, o_ref):
        o_ref[...] = jnp.dot(x_ref[...], w_ref[...],
                             preferred_element_type=jnp.float32)

    return pl.pallas_call(
        body,
        grid=(n // ROWS,),
        in_specs=[pl.BlockSpec((ROWS, din), lambda i: (i, 0)),
                  pl.BlockSpec((din, dout), lambda i: (0, 0))],
        out_specs=pl.BlockSpec((ROWS, dout), lambda i: (i, 0)),
        out_shape=jax.ShapeDtypeStruct((n, dout), jnp.float32),
    )(x, W)


def _tc_scale_split(p1, cnta2d, cntb2d):
    def body(p_ref, ca_ref, cb_ref, a_ref, b_ref):
        dinv = lax.rsqrt(ca_ref[...] + cb_ref[...] + 1.0)
        y = p_ref[...] * dinv
        a_ref[...] = y[:, :128]
        b_ref[...] = y[:, 128:]

    cb = pl.BlockSpec((ROWS, 1), lambda i: (i, 0))
    return pl.pallas_call(
        body,
        grid=(N // ROWS,),
        in_specs=[pl.BlockSpec((ROWS, D_HID), lambda i: (i, 0)), cb, cb],
        out_specs=[pl.BlockSpec((ROWS, 128), lambda i: (i, 0)),
                   pl.BlockSpec((ROWS, 128), lambda i: (i, 0))],
        out_shape=[jax.ShapeDtypeStruct((N, 128), jnp.float32),
                   jax.ShapeDtypeStruct((N, 128), jnp.float32)],
    )(p1, cnta2d, cntb2d)


def _tc_gcn2(acc1a, acc1b, y1a, y1b, cnta2d, cntb2d, b1r, W2):
    def body(aa, ab, ya, yb, ca_ref, cb_ref, b_ref, w_ref, o_ref):
        dinv = lax.rsqrt(ca_ref[...] + cb_ref[...] + 1.0)
        h1a = jnp.maximum(dinv * (aa[...] + ya[...]) + b_ref[:, :128], 0.0)
        h1b = jnp.maximum(dinv * (ab[...] + yb[...]) + b_ref[:, 128:], 0.0)
        p = (jnp.dot(h1a, w_ref[:128, :], preferred_element_type=jnp.float32)
             + jnp.dot(h1b, w_ref[128:, :],
                       preferred_element_type=jnp.float32))
        o_ref[...] = dinv * p

    rb = pl.BlockSpec((ROWS, 128), lambda i: (i, 0))
    cb = pl.BlockSpec((ROWS, 1), lambda i: (i, 0))
    return pl.pallas_call(
        body,
        grid=(N // ROWS,),
        in_specs=[rb, rb, rb, rb, cb, cb,
                  pl.BlockSpec((1, D_HID), lambda i: (0, 0)),
                  pl.BlockSpec((D_HID, D_OUT), lambda i: (0, 0))],
        out_specs=rb,
        out_shape=jax.ShapeDtypeStruct((N, D_OUT), jnp.float32),
    )(acc1a, acc1b, y1a, y1b, cnta2d, cntb2d, b1r, W2)


def _tc_qkvs(acc2a, acc2b, y2, cnta2d, cntb2d, b2r, Wq, bqr, Wk, bkr,
             Wv, bvr, Ws, bsr):
    def body(aa, ab, y_ref, ca_ref, cb_ref, b2_ref, wq, bq, wk, bk, wv, bv,
             ws, bs, q_ref, k_ref, v_ref, s_ref):
        dinv = lax.rsqrt(ca_ref[...] + cb_ref[...] + 1.0)
        h2 = dinv * (aa[...] + ab[...] + y_ref[...]) + b2_ref[...]
        q_ref[...] = jnp.dot(h2, wq[...],
                             preferred_element_type=jnp.float32) + bq[...]
        k_ref[...] = jnp.dot(h2, wk[...],
                             preferred_element_type=jnp.float32) + bk[...]
        v_ref[...] = jnp.dot(h2, wv[...],
                             preferred_element_type=jnp.float32) + bv[...]
        s_ref[...] = jnp.dot(h2, ws[...],
                             preferred_element_type=jnp.float32) + bs[...]

    rb = pl.BlockSpec((ROWS, 128), lambda i: (i, 0))
    cb = pl.BlockSpec((ROWS, 1), lambda i: (i, 0))
    wb = pl.BlockSpec((D_OUT, D_OUT), lambda i: (0, 0))
    bb = pl.BlockSpec((1, D_OUT), lambda i: (0, 0))
    return pl.pallas_call(
        body,
        grid=(N // ROWS,),
        in_specs=[rb, rb, rb, cb, cb, bb, wb, bb, wb, bb, wb, bb, wb, bb],
        out_specs=[rb, rb, rb, rb],
        out_shape=[jax.ShapeDtypeStruct((N, D_OUT), jnp.float32)
                   for _ in range(4)],
    )(acc2a, acc2b, y2, cnta2d, cntb2d, b2r, Wq, bqr, Wk, bkr, Wv, bvr,
      Ws, bsr)


def _tc_gmax(partials):
    scale = 1.0 / (128.0 ** 0.5)

    def body(p_ref, o_ref, m_ref):
        i = pl.program_id(0)
        s = jnp.sum(p_ref[...], axis=1, keepdims=True) * scale
        o_ref[...] = s

        @pl.when(i == 0)
        def _():
            m_ref[...] = jnp.full((1, 128), -jnp.inf, jnp.float32)
        m_ref[...] = jnp.maximum(m_ref[...], jnp.max(s))

    return pl.pallas_call(
        body,
        grid=(E // 2000,),
        in_specs=[pl.BlockSpec((2000, 16), lambda i: (i, 0))],
        out_specs=[pl.BlockSpec((2000, 1), lambda i: (i, 0)),
                   pl.BlockSpec((1, 128), lambda i: (0, 0))],
        out_shape=[jax.ShapeDtypeStruct((E, 1), jnp.float32),
                   jax.ShapeDtypeStruct((1, 128), jnp.float32)],
    )(partials)


def _tc_final(agga, aggb, dena2d, denb2d, sroot):
    def body(aa, ab, da, db, s_ref, o_ref):
        den = da[...] + db[...] + 1e-16
        o_ref[...] = (aa[...] + ab[...]) / den + s_ref[...]

    rb = pl.BlockSpec((ROWS, 128), lambda i: (i, 0))
    cb = pl.BlockSpec((ROWS, 1), lambda i: (i, 0))
    return pl.pallas_call(
        body,
        grid=(N // ROWS,),
        in_specs=[rb, rb, cb, cb, rb],
        out_specs=rb,
        out_shape=jax.ShapeDtypeStruct((N, D_OUT), jnp.float32),
    )(agga, aggb, dena2d, denb2d, sroot)


def kernel(x, edge_index, W1, b1, W2, b2, Wq, bq, Wk, bk, Wv, bv, Ws, bs):
    src = edge_index[0]
    dst = edge_index[1]
    b1r = b1.reshape(1, D_HID)

    cnt_a, cnt_b = _sc_histogram(dst)
    p1 = _tc_matmul(x, W1)
    ca2d = cnt_a.reshape(N, 1)
    cb2d = cnt_b.reshape(N, 1)
    y1a, y1b = _tc_scale_split(p1, ca2d, cb2d)
    acc1a, acc1b = _sc_scatter_rows((y1a, y1b), src, dst, "feat")
    y2 = _tc_gcn2(acc1a, acc1b, y1a, y1b, ca2d, cb2d, b1r, W2)
    acc2a, acc2b = _sc_scatter_rows((y2,), src, dst, "edge")
    q, kk, v, sroot = _tc_qkvs(
        acc2a, acc2b, y2, ca2d, cb2d, b2.reshape(1, D_OUT),
        Wq, bq.reshape(1, D_OUT), Wk, bk.reshape(1, D_OUT),
        Wv, bv.reshape(1, D_OUT), Ws, bs.reshape(1, D_OUT))
    partials = _sc_scores(q, kk, src, dst)
    scores2d, gmax = _tc_gmax(partials)
    agga, aggb, dena, denb = _sc_attn_scatter(
        v, scores2d.reshape(E), gmax.reshape(128), src, dst)
    out = _tc_final(agga, aggb, dena.reshape(N, 1), denb.reshape(N, 1),
                    sroot)
    return out


# speedup vs baseline: 1.0026x; 1.0026x over previous
import dataclasses
import functools

import jax
import jax.numpy as jnp
from jax import lax
from jax.experimental import pallas as pl
from jax.experimental.pallas import tpu as pltpu
from jax.experimental.pallas import tpu_sc as plsc

N = 10000
E = 160000
D_IN = 256
D_HID = 256
D_OUT = 128

NC = 2
NS = 16
CHUNK = 128
NCHUNKS = E // CHUNK
HALF = NCHUNKS // 2
RC = 128
NRC = N // RC
NREM = N - NRC * RC
NRLOOP = (NRC + NS - 1) // NS

_mesh = functools.partial(
    plsc.VectorSubcoreMesh, core_axis_name="c", subcore_axis_name="s")


def _zero_vec(ref, n):
    z = jnp.zeros((16,), jnp.float32)

    @pl.loop(0, n // 16)
    def _(i):
        ref[pl.ds(i * 16, 16)] = z


def _fill_ones(ref, n):
    o = jnp.ones((16,), jnp.float32)

    @pl.loop(0, n // 16)
    def _(i):
        ref[pl.ds(i * 16, 16)] = o


def _zero_rows(ref):
    z = jnp.zeros((16,), jnp.float32)

    @pl.loop(0, RC)
    def _(r):
        @pl.loop(0, 8)
        def _(c):
            ref[r, pl.ds(c * 16, 16)] = z


def _rows_phase(sid, fn):
    @pl.loop(0, NRLOOP)
    def _(it):
        rc = sid + it * NS

        @pl.when(rc < NRC)
        def _():
            fn(rc * RC, RC)

    @pl.when(sid == 0)
    def _():
        fn(NRC * RC, NREM)


def _edge_pipeline(cid, sid, mode, start_idx, wait_idx, start_body,
                   finish_body):
    if mode == "feat":
        nloop = (NCHUNKS + NS - 1) // NS

        def chunk_of(j):
            return sid + j * NS

        def limit():
            return NCHUNKS
    else:
        nloop = (HALF + NS - 1) // NS

        def chunk_of(j):
            return cid * HALF + sid + j * NS

        def limit():
            return (cid + 1) * HALF

    npair = (nloop + 1) // 2

    for b in (0, 1):
        @pl.when(chunk_of(b) < limit())
        def _(b=b):
            start_idx(b, b)

    @pl.loop(0, npair)
    def _(j2):
        j0 = j2 * 2
        for b in (0, 1):
            j = j0 + b

            @pl.when(chunk_of(j) < limit())
            def _(b=b, j=j):
                wait_idx(j, b)
                start_body(j, b)
        for b in (0, 1):
            j = j0 + b

            @pl.when(chunk_of(j) < limit())
            def _(b=b, j=j):
                finish_body(j, b)

                @pl.when(chunk_of(j + 2) < limit())
                def _():
                    start_idx(j + 2, b)

    return chunk_of


def _sc_histogram(dst):
    out_type = (jax.ShapeDtypeStruct((N,), jnp.float32),
                jax.ShapeDtypeStruct((N,), jnp.float32))

    @functools.partial(
        pl.kernel,
        out_type=out_type,
        mesh=_mesh(),
        scratch_types=[
            pltpu.VMEM((CHUNK,), jnp.float32),
            pltpu.VMEM((CHUNK,), jnp.int32),
            pltpu.VMEM((CHUNK,), jnp.int32),
            pltpu.VMEM((2000,), jnp.float32),
            pltpu.VMEM((N,), jnp.float32),
            pltpu.SemaphoreType.DMA,
            pltpu.SemaphoreType.DMA,
            pltpu.VMEM_SHARED((N,), jnp.float32),
        ],
    )
    def k(dst_hbm, cnt_a, cnt_b, ones_v, idx0, idx1, zvec, stage,
          s0, s1, acc_s):
        cid = lax.axis_index("c")
        sid = lax.axis_index("s")
        idx = (idx0, idx1)
        sem = (s0, s1)

        _fill_ones(ones_v, CHUNK)

        @pl.when(sid == 0)
        def _():
            _zero_vec(zvec, 2000)

            @pl.loop(0, 5)
            def _(j):
                pltpu.sync_copy(zvec, acc_s.at[pl.ds(j * 2000, 2000)])

        plsc.subcore_barrier()

        def start_idx(j, b):
            base = (cid * HALF + sid + j * NS) * CHUNK
            pltpu.async_copy(dst_hbm.at[pl.ds(base, CHUNK)], idx[b], sem[b])

        def wait_idx(j, b):
            base = (cid * HALF + sid + j * NS) * CHUNK
            pltpu.make_async_copy(dst_hbm.at[pl.ds(base, CHUNK)], idx[b],
                                  sem[b]).wait()

        def start_body(j, b):
            pass

        def finish_body(j, b):
            pltpu.sync_copy(ones_v, acc_s.at[idx[b]], add=True)

        _edge_pipeline(cid, sid, "edge", start_idx, wait_idx, start_body,
                       finish_body)

        plsc.subcore_barrier()

        @pl.when(sid == 0)
        def _():
            pltpu.sync_copy(acc_s, stage)

            @pl.when(cid == 0)
            def _():
                pltpu.sync_copy(stage, cnt_a)

            @pl.when(cid == 1)
            def _():
                pltpu.sync_copy(stage, cnt_b)

    return k(dst)


def _sc_scatter_rows(tables, src, dst, mode):
    out_type = tuple(jax.ShapeDtypeStruct((N, 128), jnp.float32)
                     for _ in range(2))

    @functools.partial(
        pl.kernel,
        out_type=out_type,
        mesh=_mesh(),
        scratch_types=[
            pltpu.VMEM((CHUNK,), jnp.int32),
            pltpu.VMEM((CHUNK,), jnp.int32),
            pltpu.VMEM((CHUNK,), jnp.int32),
            pltpu.VMEM((CHUNK,), jnp.int32),
            pltpu.VMEM((CHUNK, 128), jnp.float32),
            pltpu.VMEM((CHUNK, 128), jnp.float32),
            pltpu.SemaphoreType.DMA,
            pltpu.SemaphoreType.DMA,
            pltpu.SemaphoreType.DMA,
            pltpu.SemaphoreType.DMA,
            pltpu.SemaphoreType.DMA,
            pltpu.SemaphoreType.DMA,
            pltpu.VMEM_SHARED((N, 128), jnp.float32),
        ],
    )
    def k(*refs):
        if mode == "feat":
            ta, tb, src_hbm, dst_hbm, out_a, out_b = refs[:6]
            nin = 6
        else:
            tab, src_hbm, dst_hbm, out_a, out_b = refs[:5]
            nin = 5
        (sidx0, sidx1, didx0, didx1, rows0, rows1,
         ss0, ss1, sd0, sd1, sg0, sg1, acc_s) = refs[nin:]
        sidx = (sidx0, sidx1)
        didx = (didx0, didx1)
        rows = (rows0, rows1)
        ssem = (ss0, ss1)
        dsem = (sd0, sd1)
        gsem = (sg0, sg1)
        cid = lax.axis_index("c")
        sid = lax.axis_index("s")

        _zero_rows(rows0)
        _rows_phase(sid, lambda r0, nr: pltpu.sync_copy(
            rows0.at[pl.ds(0, nr)], acc_s.at[pl.ds(r0, nr)]))

        plsc.subcore_barrier()

        if mode == "feat":
            def base_of(j):
                return (sid + j * NS) * CHUNK
        else:
            def base_of(j):
                return (cid * HALF + sid + j * NS) * CHUNK

        def start_idx(j, b):
            base = base_of(j)
            pltpu.async_copy(src_hbm.at[pl.ds(base, CHUNK)], sidx[b],
                             ssem[b])
            pltpu.async_copy(dst_hbm.at[pl.ds(base, CHUNK)], didx[b],
                             dsem[b])

        def wait_idx(j, b):
            base = base_of(j)
            pltpu.make_async_copy(src_hbm.at[pl.ds(base, CHUNK)], sidx[b],
                                  ssem[b]).wait()
            pltpu.make_async_copy(dst_hbm.at[pl.ds(base, CHUNK)], didx[b],
                                  dsem[b]).wait()

        def start_body(j, b):
            if mode == "feat":
                @pl.when(cid == 0)
                def _():
                    pltpu.async_copy(ta.at[sidx[b]], rows[b], gsem[b])

                @pl.when(cid == 1)
                def _():
                    pltpu.async_copy(tb.at[sidx[b]], rows[b], gsem[b])
            else:
                pltpu.async_copy(tab.at[sidx[b]], rows[b], gsem[b])

        def finish_body(j, b):
            first = ta if mode == "feat" else tab
            pltpu.make_async_copy(first.at[sidx[b]], rows[b],
                                  gsem[b]).wait()
            pltpu.sync_copy(rows[b], acc_s.at[didx[b]], add=True)

        _edge_pipeline(cid, sid, mode, start_idx, wait_idx, start_body,
                       finish_body)

        plsc.subcore_barrier()

        def wb(r0, nr):
            pltpu.sync_copy(acc_s.at[pl.ds(r0, nr)], rows0.at[pl.ds(0, nr)])

            @pl.when(cid == 0)
            def _():
                pltpu.sync_copy(rows0.at[pl.ds(0, nr)],
                                out_a.at[pl.ds(r0, nr)])

            @pl.when(cid == 1)
            def _():
                pltpu.sync_copy(rows0.at[pl.ds(0, nr)],
                                out_b.at[pl.ds(r0, nr)])

        _rows_phase(sid, wb)

    if mode == "feat":
        return k(tables[0], tables[1], src, dst)
    return k(tables[0], src, dst)


def _sc_scores(q, kk, src, dst):
    @functools.partial(
        pl.kernel,
        out_type=jax.ShapeDtypeStruct((E, 16), jnp.float32),
        mesh=_mesh(),
        scratch_types=[
            pltpu.VMEM((CHUNK,), jnp.int32),
            pltpu.VMEM((CHUNK,), jnp.int32),
            pltpu.VMEM((CHUNK,), jnp.int32),
            pltpu.VMEM((CHUNK,), jnp.int32),
            pltpu.VMEM((CHUNK, 128), jnp.float32),
            pltpu.VMEM((CHUNK, 128), jnp.float32),
            pltpu.VMEM((CHUNK, 128), jnp.float32),
            pltpu.VMEM((CHUNK, 128), jnp.float32),
            pltpu.VMEM((CHUNK, 16), jnp.float32),
            pltpu.VMEM((CHUNK, 16), jnp.float32),
            pltpu.SemaphoreType.DMA,
            pltpu.SemaphoreType.DMA,
            pltpu.SemaphoreType.DMA,
            pltpu.SemaphoreType.DMA,
            pltpu.SemaphoreType.DMA,
            pltpu.SemaphoreType.DMA,
            pltpu.SemaphoreType.DMA,
            pltpu.SemaphoreType.DMA,
        ],
    )
    def k(q_hbm, k_hbm, src_hbm, dst_hbm, sc_hbm,
          sidx0, sidx1, didx0, didx1, rq0, rq1, rk0, rk1, sb0, sb1,
          ss0, ss1, sd0, sd1, sq0, sq1, sk0, sk1):
        sidx = (sidx0, sidx1)
        didx = (didx0, didx1)
        rq = (rq0, rq1)
        rk = (rk0, rk1)
        sbuf = (sb0, sb1)
        ssem = (ss0, ss1)
        dsem = (sd0, sd1)
        qsem = (sq0, sq1)
        ksem = (sk0, sk1)
        cid = lax.axis_index("c")
        sid = lax.axis_index("s")

        def base_of(j):
            return (cid * HALF + sid + j * NS) * CHUNK

        def start_idx(j, b):
            base = base_of(j)
            pltpu.async_copy(src_hbm.at[pl.ds(base, CHUNK)], sidx[b],
                             ssem[b])
            pltpu.async_copy(dst_hbm.at[pl.ds(base, CHUNK)], didx[b],
                             dsem[b])

        def wait_idx(j, b):
            base = base_of(j)
            pltpu.make_async_copy(src_hbm.at[pl.ds(base, CHUNK)], sidx[b],
                                  ssem[b]).wait()
            pltpu.make_async_copy(dst_hbm.at[pl.ds(base, CHUNK)], didx[b],
                                  dsem[b]).wait()

        def start_body(j, b):
            pltpu.async_copy(q_hbm.at[didx[b]], rq[b], qsem[b])
            pltpu.async_copy(k_hbm.at[sidx[b]], rk[b], ksem[b])

        def finish_body(j, b):
            base = base_of(j)
            pltpu.make_async_copy(q_hbm.at[didx[b]], rq[b], qsem[b]).wait()
            pltpu.make_async_copy(k_hbm.at[sidx[b]], rk[b], ksem[b]).wait()

            @pl.loop(0, CHUNK)
            def _(r):
                acc = rq[b][r, pl.ds(0, 16)] * rk[b][r, pl.ds(0, 16)]
                for c in range(1, 8):
                    sl = pl.ds(c * 16, 16)
                    acc = acc + rq[b][r, sl] * rk[b][r, sl]
                sbuf[b][r, pl.ds(0, 16)] = acc

            pltpu.sync_copy(sbuf[b], sc_hbm.at[pl.ds(base, CHUNK)])

        _edge_pipeline(cid, sid, "edge", start_idx, wait_idx, start_body,
                       finish_body)

    return k(q, kk, src, dst)


def _sc_attn_scatter(v, scores, gmax, src, dst):
    out_type = (jax.ShapeDtypeStruct((N, 128), jnp.float32),
                jax.ShapeDtypeStruct((N, 128), jnp.float32),
                jax.ShapeDtypeStruct((N,), jnp.float32),
                jax.ShapeDtypeStruct((N,), jnp.float32))

    @functools.partial(
        pl.kernel,
        out_type=out_type,
        mesh=_mesh(),
        scratch_types=[
            pltpu.VMEM((CHUNK,), jnp.int32),
            pltpu.VMEM((CHUNK,), jnp.int32),
            pltpu.VMEM((CHUNK,), jnp.int32),
            pltpu.VMEM((CHUNK,), jnp.int32),
            pltpu.VMEM((CHUNK, 128), jnp.float32),
            pltpu.VMEM((CHUNK, 128), jnp.float32),
            pltpu.VMEM((CHUNK,), jnp.float32),
            pltpu.VMEM((CHUNK,), jnp.float32),
            pltpu.VMEM((128,), jnp.float32),
            pltpu.VMEM((2000,), jnp.float32),
            pltpu.SemaphoreType.DMA,
            pltpu.SemaphoreType.DMA,
            pltpu.SemaphoreType.DMA,
            pltpu.SemaphoreType.DMA,
            pltpu.SemaphoreType.DMA,
            pltpu.SemaphoreType.DMA,
            pltpu.SemaphoreType.DMA,
            pltpu.SemaphoreType.DMA,
            pltpu.VMEM_SHARED((N, 128), jnp.float32),
            pltpu.VMEM_SHARED((N,), jnp.float32),
        ],
    )
    def k(v_hbm, sc_hbm, gmax_hbm, src_hbm, dst_hbm,
          agg_a, agg_b, den_a, den_b,
          sidx0, sidx1, didx0, didx1, rows0, rows1, ev0, ev1, gbuf, zvec,
          ss0, ss1, sd0, sd1, sv0, sv1, sg0, sg1, acc_s, den_s):
        sidx = (sidx0, sidx1)
        didx = (didx0, didx1)
        rows = (rows0, rows1)
        ev = (ev0, ev1)
        ssem = (ss0, ss1)
        dsem = (sd0, sd1)
        vsem = (sv0, sv1)
        gsem = (sg0, sg1)
        cid = lax.axis_index("c")
        sid = lax.axis_index("s")

        pltpu.sync_copy(gmax_hbm, gbuf)

        _zero_rows(rows0)
        _rows_phase(sid, lambda r0, nr: pltpu.sync_copy(
            rows0.at[pl.ds(0, nr)], acc_s.at[pl.ds(r0, nr)]))

        @pl.when(sid == 0)
        def _():
            _zero_vec(zvec, 2000)

            @pl.loop(0, 5)
            def _(j):
                pltpu.sync_copy(zvec, den_s.at[pl.ds(j * 2000, 2000)])

        plsc.subcore_barrier()

        def base_of(j):
            return (cid * HALF + sid + j * NS) * CHUNK

        def start_idx(j, b):
            base = base_of(j)
            pltpu.async_copy(src_hbm.at[pl.ds(base, CHUNK)], sidx[b],
                             ssem[b])
            pltpu.async_copy(dst_hbm.at[pl.ds(base, CHUNK)], didx[b],
                             dsem[b])
            pltpu.async_copy(sc_hbm.at[pl.ds(base, CHUNK)], ev[b], vsem[b])

        def wait_idx(j, b):
            base = base_of(j)
            pltpu.make_async_copy(src_hbm.at[pl.ds(base, CHUNK)], sidx[b],
                                  ssem[b]).wait()
            pltpu.make_async_copy(dst_hbm.at[pl.ds(base, CHUNK)], didx[b],
                                  dsem[b]).wait()
            pltpu.make_async_copy(sc_hbm.at[pl.ds(base, CHUNK)], ev[b],
                                  vsem[b]).wait()

        def start_body(j, b):
            pltpu.async_copy(v_hbm.at[sidx[b]], rows[b], gsem[b])

        def finish_body(j, b):
            gm = gbuf[pl.ds(0, 16)][0]
            @pl.loop(0, CHUNK // 16)
            def _(i):
                s = ev[b][pl.ds(i * 16, 16)]
                ev[b][pl.ds(i * 16, 16)] = jnp.exp(s - gm)

            pltpu.make_async_copy(v_hbm.at[sidx[b]], rows[b],
                                  gsem[b]).wait()

            @pl.loop(0, CHUNK // 16)
            def _(g):
                evec = ev[b][pl.ds(g * 16, 16)]
                for kk_ in range(16):
                    er = evec[kk_]
                    r = g * 16 + kk_
                    for c in range(8):
                        sl = pl.ds(c * 16, 16)
                        rows[b][r, sl] = rows[b][r, sl] * er

            pltpu.sync_copy(rows[b], acc_s.at[didx[b]], add=True)
            pltpu.sync_copy(ev[b], den_s.at[didx[b]], add=True)

        _edge_pipeline(cid, sid, "edge", start_idx, wait_idx, start_body,
                       finish_body)

        plsc.subcore_barrier()

        def wb(r0, nr):
            pltpu.sync_copy(acc_s.at[pl.ds(r0, nr)], rows0.at[pl.ds(0, nr)])

            @pl.when(cid == 0)
            def _():
                pltpu.sync_copy(rows0.at[pl.ds(0, nr)],
                                agg_a.at[pl.ds(r0, nr)])

            @pl.when(cid == 1)
            def _():
                pltpu.sync_copy(rows0.at[pl.ds(0, nr)],
                                agg_b.at[pl.ds(r0, nr)])

        _rows_phase(sid, wb)

        @pl.when(sid == 0)
        def _():
            @pl.loop(0, 5)
            def _(j):
                pltpu.sync_copy(den_s.at[pl.ds(j * 2000, 2000)], zvec)

                @pl.when(cid == 0)
                def _():
                    pltpu.sync_copy(zvec, den_a.at[pl.ds(j * 2000, 2000)])

                @pl.when(cid == 1)
                def _():
                    pltpu.sync_copy(zvec, den_b.at[pl.ds(j * 2000, 2000)])

    return k(v, scores, gmax, src, dst)


ROWS = 1000


def _tc_matmul(x, W):
    n, din = x.shape
    dout = W.shape[1]

    def body(x_ref, w_ref, o_ref):
        o_ref[...] = jnp.dot(x_ref[...], w_ref[...],
                             preferred_element_type=jnp.float32)

    return pl.pallas_call(
        body,
        grid=(n // ROWS,),
        in_specs=[pl.BlockSpec((ROWS, din), lambda i: (i, 0)),
                  pl.BlockSpec((din, dout), lambda i: (0, 0))],
        out_specs=pl.BlockSpec((ROWS, dout), lambda i: (i, 0)),
        out_shape=jax.ShapeDtypeStruct((n, dout), jnp.float32),
    )(x, W)


def _tc_scale_split(x, W1, cnta2d, cntb2d):
    def body(x_ref, w_ref, ca_ref, cb_ref, a_ref, b_ref):
        p = jnp.dot(x_ref[...], w_ref[...],
                    preferred_element_type=jnp.float32)
        dinv = lax.rsqrt(ca_ref[...] + cb_ref[...] + 1.0)
        y = p * dinv
        a_ref[...] = y[:, :128]
        b_ref[...] = y[:, 128:]

    cb = pl.BlockSpec((ROWS, 1), lambda i: (i, 0))
    return pl.pallas_call(
        body,
        grid=(N // ROWS,),
        in_specs=[pl.BlockSpec((ROWS, D_IN), lambda i: (i, 0)),
                  pl.BlockSpec((D_IN, D_HID), lambda i: (0, 0)), cb, cb],
        out_specs=[pl.BlockSpec((ROWS, 128), lambda i: (i, 0)),
                   pl.BlockSpec((ROWS, 128), lambda i: (i, 0))],
        out_shape=[jax.ShapeDtypeStruct((N, 128), jnp.float32),
                   jax.ShapeDtypeStruct((N, 128), jnp.float32)],
    )(x, W1, cnta2d, cntb2d)


def _tc_gcn2(acc1a, acc1b, y1a, y1b, cnta2d, cntb2d, b1r, W2):
    def body(aa, ab, ya, yb, ca_ref, cb_ref, b_ref, w_ref, o_ref):
        dinv = lax.rsqrt(ca_ref[...] + cb_ref[...] + 1.0)
        h1a = jnp.maximum(dinv * (aa[...] + ya[...]) + b_ref[:, :128], 0.0)
        h1b = jnp.maximum(dinv * (ab[...] + yb[...]) + b_ref[:, 128:], 0.0)
        p = (jnp.dot(h1a, w_ref[:128, :], preferred_element_type=jnp.float32)
             + jnp.dot(h1b, w_ref[128:, :],
                       preferred_element_type=jnp.float32))
        o_ref[...] = dinv * p

    rb = pl.BlockSpec((ROWS, 128), lambda i: (i, 0))
    cb = pl.BlockSpec((ROWS, 1), lambda i: (i, 0))
    return pl.pallas_call(
        body,
        grid=(N // ROWS,),
        in_specs=[rb, rb, rb, rb, cb, cb,
                  pl.BlockSpec((1, D_HID), lambda i: (0, 0)),
                  pl.BlockSpec((D_HID, D_OUT), lambda i: (0, 0))],
        out_specs=rb,
        out_shape=jax.ShapeDtypeStruct((N, D_OUT), jnp.float32),
    )(acc1a, acc1b, y1a, y1b, cnta2d, cntb2d, b1r, W2)


def _tc_qkvs(acc2a, acc2b, y2, cnta2d, cntb2d, b2r, Wq, bqr, Wk, bkr,
             Wv, bvr, Ws, bsr):
    def body(aa, ab, y_ref, ca_ref, cb_ref, b2_ref, wq, bq, wk, bk, wv, bv,
             ws, bs, q_ref, k_ref, v_ref, s_ref):
        dinv = lax.rsqrt(ca_ref[...] + cb_ref[...] + 1.0)
        h2 = dinv * (aa[...] + ab[...] + y_ref[...]) + b2_ref[...]
        q_ref[...] = jnp.dot(h2, wq[...],
                             preferred_element_type=jnp.float32) + bq[...]
        k_ref[...] = jnp.dot(h2, wk[...],
                             preferred_element_type=jnp.float32) + bk[...]
        v_ref[...] = jnp.dot(h2, wv[...],
                             preferred_element_type=jnp.float32) + bv[...]
        s_ref[...] = jnp.dot(h2, ws[...],
                             preferred_element_type=jnp.float32) + bs[...]

    rb = pl.BlockSpec((ROWS, 128), lambda i: (i, 0))
    cb = pl.BlockSpec((ROWS, 1), lambda i: (i, 0))
    wb = pl.BlockSpec((D_OUT, D_OUT), lambda i: (0, 0))
    bb = pl.BlockSpec((1, D_OUT), lambda i: (0, 0))
    return pl.pallas_call(
        body,
        grid=(N // ROWS,),
        in_specs=[rb, rb, rb, cb, cb, bb, wb, bb, wb, bb, wb, bb, wb, bb],
        out_specs=[rb, rb, rb, rb],
        out_shape=[jax.ShapeDtypeStruct((N, D_OUT), jnp.float32)
                   for _ in range(4)],
    )(acc2a, acc2b, y2, cnta2d, cntb2d, b2r, Wq, bqr, Wk, bkr, Wv, bvr,
      Ws, bsr)


def _tc_gmax(partials):
    scale = 1.0 / (128.0 ** 0.5)

    def body(p_ref, o_ref, m_ref):
        i = pl.program_id(0)
        s = jnp.sum(p_ref[...], axis=1, keepdims=True) * scale
        o_ref[...] = s

        @pl.when(i == 0)
        def _():
            m_ref[...] = jnp.full((1, 128), -jnp.inf, jnp.float32)
        m_ref[...] = jnp.maximum(m_ref[...], jnp.max(s))

    return pl.pallas_call(
        body,
        grid=(E // 2000,),
        in_specs=[pl.BlockSpec((2000, 16), lambda i: (i, 0))],
        out_specs=[pl.BlockSpec((2000, 1), lambda i: (i, 0)),
                   pl.BlockSpec((1, 128), lambda i: (0, 0))],
        out_shape=[jax.ShapeDtypeStruct((E, 1), jnp.float32),
                   jax.ShapeDtypeStruct((1, 128), jnp.float32)],
    )(partials)


def _tc_final(agga, aggb, dena2d, denb2d, sroot):
    def body(aa, ab, da, db, s_ref, o_ref):
        den = da[...] + db[...] + 1e-16
        o_ref[...] = (aa[...] + ab[...]) / den + s_ref[...]

    rb = pl.BlockSpec((ROWS, 128), lambda i: (i, 0))
    cb = pl.BlockSpec((ROWS, 1), lambda i: (i, 0))
    return pl.pallas_call(
        body,
        grid=(N // ROWS,),
        in_specs=[rb, rb, cb, cb, rb],
        out_specs=rb,
        out_shape=jax.ShapeDtypeStruct((N, D_OUT), jnp.float32),
    )(agga, aggb, dena2d, denb2d, sroot)


def kernel(x, edge_index, W1, b1, W2, b2, Wq, bq, Wk, bk, Wv, bv, Ws, bs):
    src = edge_index[0]
    dst = edge_index[1]
    b1r = b1.reshape(1, D_HID)

    cnt_a, cnt_b = _sc_histogram(dst)
    ca2d = cnt_a.reshape(N, 1)
    cb2d = cnt_b.reshape(N, 1)
    y1a, y1b = _tc_scale_split(x, W1, ca2d, cb2d)
    acc1a, acc1b = _sc_scatter_rows((y1a, y1b), src, dst, "feat")
    y2 = _tc_gcn2(acc1a, acc1b, y1a, y1b, ca2d, cb2d, b1r, W2)
    acc2a, acc2b = _sc_scatter_rows((y2,), src, dst, "edge")
    q, kk, v, sroot = _tc_qkvs(
        acc2a, acc2b, y2, ca2d, cb2d, b2.reshape(1, D_OUT),
        Wq, bq.reshape(1, D_OUT), Wk, bk.reshape(1, D_OUT),
        Wv, bv.reshape(1, D_OUT), Ws, bs.reshape(1, D_OUT))
    partials = _sc_scores(q, kk, src, dst)
    scores2d, gmax = _tc_gmax(partials)
    agga, aggb, dena, denb = _sc_attn_scatter(
        v, scores2d.reshape(E), gmax.reshape(128), src, dst)
    out = _tc_final(agga, aggb, dena.reshape(N, 1), denb.reshape(N, 1),
                    sroot)
    return out
